# baseline (speedup 1.0000x reference)
import functools

import jax
import jax.numpy as jnp
import numpy as np
from jax import lax
from jax.experimental import pallas as pl
from jax.experimental.pallas import tpu as pltpu
from jax.experimental.pallas import tpu_sc as plsc

_B = 1024
_NBITS = 22
_NCODES = 23
_LTAB = 131072
_ROWW = 384
_NWORK = 32
_SPW = _B // _NWORK


def _jmap_word(wi):
    g, w = wi // 4, wi % 4
    return g if (g < 7 and w < 2) else g + 7


def _bit_weights():
    wmat = np.zeros((512, _NCODES), np.float32)
    for j in range(_NCODES):
        for i in range(_NBITS):
            wmat[6 + _NBITS * j + i, j] = float(1 << i)
    return jnp.asarray(wmat)


def _sign_expand():
    emat = np.zeros((_NCODES, 128), np.float32)
    for k in range(128):
        emat[_jmap_word(k // 2), k] = 1.0
    return jnp.asarray(emat)


def _decode_body(x_ref, w_ref, e_ref, codes_ref, sgn_ref):
    xf = x_ref[...].astype(jnp.float32)
    vals = jnp.dot(xf, w_ref[...], preferred_element_type=jnp.float32)
    codes_ref[...] = vals.astype(jnp.int32) & (_LTAB - 1)
    sgnv = jnp.where(vals > jnp.float32(_LTAB), -1.0, 1.0).astype(jnp.float32)
    sgn_ref[...] = jnp.dot(sgnv, e_ref[...], preferred_element_type=jnp.float32)


def _decode(x):
    return pl.pallas_call(
        _decode_body,
        out_shape=(
            jax.ShapeDtypeStruct((_B, _NCODES), jnp.int32),
            jax.ShapeDtypeStruct((_B, 128), jnp.float32),
        ),
    )(x, _bit_weights(), _sign_expand())


def _piece(j):
    if j < 7:
        return 0, 4, 8 * j
    if j < 14:
        return 4, 4, 8 * (j - 7) + 4
    return 0, 8, 8 * (j - 7)


def _sc_gather_body(codes_hbm, data_hbm, out_hbm, idx_v, rows_v, asm_sh, sem, sem2):
    cid = lax.axis_index("c")
    sid = lax.axis_index("s")
    wid = sid * 2 + cid
    base = wid * _SPW
    pltpu.sync_copy(codes_hbm.at[pl.ds(base, _SPW)], idx_v)
    asm_v = asm_sh.at[sid]

    def sample_body(i, carry):
        cp = pltpu.make_async_copy(data_hbm.at[idx_v.at[i]], rows_v, sem)
        cp.start()
        cp.wait()
        cps = []
        for j in range(_NCODES):
            w0, cw, c0 = _piece(j)
            c = pltpu.make_async_copy(
                rows_v.at[j, :, :, pl.ds(w0, cw)],
                asm_v.at[:, :, pl.ds(c0, cw)],
                sem2,
            )
            c.start()
            cps.append(c)
        for c in cps:
            c.wait()
        pltpu.sync_copy(asm_v, out_hbm.at[base + i])
        return carry

    lax.fori_loop(0, _SPW, sample_body, 0, unroll=False)


@functools.cache
def _sc_gather():
    return pl.kernel(
        _sc_gather_body,
        out_type=jax.ShapeDtypeStruct((_B, 2, 48, 128), jnp.float16),
        mesh=plsc.VectorSubcoreMesh(
            core_axis_name="c", subcore_axis_name="s", num_cores=2, num_subcores=16
        ),
        scratch_types=[
            pltpu.VMEM((_SPW, _NCODES), jnp.int32),
            pltpu.VMEM((_NCODES, 2, 48, 16), jnp.float16),
            pltpu.VMEM_SHARED((16, 2, 48, 128), jnp.float16),
            pltpu.SemaphoreType.DMA,
            pltpu.SemaphoreType.DMA,
        ],
        compiler_params=pltpu.CompilerParams(
            needs_layout_passes=False, use_tc_tiling_on_sc=False
        ),
    )


def _finish_body(g_ref, s_ref, o_ref):
    g = g_ref[...]
    s = s_ref[...][:, None, :]
    core = 0.5 + s * (g - 0.5)
    bs = g.shape[0]
    half = jnp.full((bs, 1, 19, 128), 0.5, jnp.float32)
    o_ref[:, :, 0:19, :] = jnp.broadcast_to(half, (bs, 2, 19, 128))
    o_ref[:, 0, 19:67, :] = core[:, 0:48]
    o_ref[:, 1, 19:67, :] = core[:, 48:96]
    o_ref[:, :, 67:126, :] = jnp.full((bs, 2, 59, 128), 0.5, jnp.float32)


def _finish(gathf, sgn):
    bs = 8
    return pl.pallas_call(
        _finish_body,
        grid=(_B // bs,),
        in_specs=[
            pl.BlockSpec((bs, 96, 128), lambda i: (i, 0, 0)),
            pl.BlockSpec((bs, 128), lambda i: (i, 0)),
        ],
        out_specs=pl.BlockSpec((bs, 2, 126, 128), lambda i: (i, 0, 0, 0)),
        out_shape=jax.ShapeDtypeStruct((_B, 2, 126, 128), jnp.float32),
    )(gathf, sgn)


def kernel(x, data):
    codes, sgn = _decode(x)
    datap = jnp.pad(data, ((0, 0), (0, 0), (0, 0), (0, 8)))
    gath16 = _sc_gather()(codes, datap)
    g32 = gath16.reshape(_B, 96, 128).astype(jnp.float32)
    return _finish(g32, sgn)


# speedup vs baseline: 1.3495x; 1.3495x over previous
import functools

import jax
import jax.numpy as jnp
import numpy as np
from jax import lax
from jax.experimental import pallas as pl
from jax.experimental.pallas import tpu as pltpu
from jax.experimental.pallas import tpu_sc as plsc

_B = 1024
_NBITS = 22
_NCODES = 23
_LTAB = 131072
_ROWW = 384
_NWORK = 32
_SPW = _B // _NWORK


def _jmap_word(wi):
    g, w = wi // 4, wi % 4
    return g if (g < 7 and w < 2) else g + 7


def _bit_weights():
    wmat = np.zeros((512, _NCODES), np.float32)
    for j in range(_NCODES):
        for i in range(_NBITS):
            wmat[6 + _NBITS * j + i, j] = float(1 << i)
    return jnp.asarray(wmat)


def _sign_expand():
    emat = np.zeros((_NCODES, 128), np.float32)
    for k in range(128):
        emat[_jmap_word(k // 2), k] = 1.0
    return jnp.asarray(emat)


def _decode_body(x_ref, w_ref, e_ref, codes_ref, sgn_ref):
    xf = x_ref[...].astype(jnp.float32)
    vals = jnp.dot(xf, w_ref[...], preferred_element_type=jnp.float32)
    codes_ref[...] = vals.astype(jnp.int32) & (_LTAB - 1)
    sgnv = jnp.where(vals > jnp.float32(_LTAB), -1.0, 1.0).astype(jnp.float32)
    sgn_ref[...] = jnp.dot(sgnv, e_ref[...], preferred_element_type=jnp.float32)


def _decode(x):
    return pl.pallas_call(
        _decode_body,
        out_shape=(
            jax.ShapeDtypeStruct((_B, _NCODES), jnp.int32),
            jax.ShapeDtypeStruct((_B, 128), jnp.float32),
        ),
    )(x, _bit_weights(), _sign_expand())


def _piece(j):
    if j < 7:
        return 0, 4, 8 * j
    if j < 14:
        return 4, 4, 8 * (j - 7) + 4
    return 0, 8, 8 * (j - 7)


def _sc_gather_body(codes_hbm, data_hbm, out_hbm, idx_v, rows_v, asm_sh, sem, sem2):
    cid = lax.axis_index("c")
    sid = lax.axis_index("s")
    wid = sid * 2 + cid
    base = wid * _SPW
    pltpu.sync_copy(codes_hbm.at[pl.ds(base, _SPW)], idx_v)
    asm_v = asm_sh.at[sid]

    def sample_body(i, carry):
        cp = pltpu.make_async_copy(data_hbm.at[idx_v.at[i]], rows_v, sem)
        cp.start()
        cp.wait()
        cps = []
        for j in range(_NCODES):
            w0, cw, c0 = _piece(j)
            c = pltpu.make_async_copy(
                rows_v.at[j, :, :, pl.ds(w0, cw)],
                asm_v.at[:, :, pl.ds(c0, cw)],
                sem2,
            )
            c.start()
            cps.append(c)
        for c in cps:
            c.wait()
        pltpu.sync_copy(asm_v, out_hbm.at[base + i])
        return carry

    lax.fori_loop(0, _SPW, sample_body, 0, unroll=False)


@functools.cache
def _sc_gather():
    return pl.kernel(
        _sc_gather_body,
        out_type=jax.ShapeDtypeStruct((_B, 2, 48, 128), jnp.float32),
        mesh=plsc.VectorSubcoreMesh(
            core_axis_name="c", subcore_axis_name="s", num_cores=2, num_subcores=16
        ),
        scratch_types=[
            pltpu.VMEM((_SPW, _NCODES), jnp.int32),
            pltpu.VMEM((_NCODES, 2, 48, 8), jnp.float32),
            pltpu.VMEM_SHARED((16, 2, 48, 128), jnp.float32),
            pltpu.SemaphoreType.DMA,
            pltpu.SemaphoreType.DMA,
        ],
        compiler_params=pltpu.CompilerParams(
            needs_layout_passes=False, use_tc_tiling_on_sc=False
        ),
    )


def _finish_body(g_ref, s_ref, o_ref):
    g = g_ref[...]
    s = s_ref[...][:, None, :]
    core = 0.5 + s * (g - 0.5)
    bs = g.shape[0]
    half = jnp.full((bs, 1, 19, 128), 0.5, jnp.float32)
    o_ref[:, :, 0:19, :] = jnp.broadcast_to(half, (bs, 2, 19, 128))
    o_ref[:, 0, 19:67, :] = core[:, 0:48]
    o_ref[:, 1, 19:67, :] = core[:, 48:96]
    o_ref[:, :, 67:126, :] = jnp.full((bs, 2, 59, 128), 0.5, jnp.float32)


def _finish(gathf, sgn):
    bs = 8
    return pl.pallas_call(
        _finish_body,
        grid=(_B // bs,),
        in_specs=[
            pl.BlockSpec((bs, 96, 128), lambda i: (i, 0, 0)),
            pl.BlockSpec((bs, 128), lambda i: (i, 0)),
        ],
        out_specs=pl.BlockSpec((bs, 2, 126, 128), lambda i: (i, 0, 0, 0)),
        out_shape=jax.ShapeDtypeStruct((_B, 2, 126, 128), jnp.float32),
    )(gathf, sgn)


def kernel(x, data):
    codes, sgn = _decode(x)
    data32 = jax.lax.optimization_barrier(data.astype(jnp.float32))
    g32 = _sc_gather()(codes, data32)
    return _finish(g32.reshape(_B, 96, 128), sgn)


# speedup vs baseline: 2.9921x; 2.2172x over previous
import functools

import jax
import jax.numpy as jnp
import numpy as np
from jax import lax
from jax.experimental import pallas as pl
from jax.experimental.pallas import tpu as pltpu
from jax.experimental.pallas import tpu_sc as plsc

_B = 1024
_NBITS = 22
_NCODES = 23
_LTAB = 131072
_ROWW = 384
_NWORK = 32
_SPW = _B // _NWORK


def _jmap_word(wi):
    g, w = wi // 4, wi % 4
    return g if (g < 7 and w < 2) else g + 7


def _bit_weights():
    wmat = np.zeros((512, _NCODES), np.float32)
    for j in range(_NCODES):
        for i in range(_NBITS):
            wmat[6 + _NBITS * j + i, j] = float(1 << i)
    return jnp.asarray(wmat)


def _sign_expand():
    emat = np.zeros((_NCODES, 128), np.float32)
    for k in range(128):
        emat[_jmap_word(k // 2), k] = 1.0
    return jnp.asarray(emat)


def _decode_body(x_ref, w_ref, e_ref, codes_ref, sgn_ref):
    xf = x_ref[...].astype(jnp.float32)
    vals = jnp.dot(xf, w_ref[...], preferred_element_type=jnp.float32)
    codes_ref[...] = vals.astype(jnp.int32) & (_LTAB - 1)
    sgnv = jnp.where(vals > jnp.float32(_LTAB), -1.0, 1.0).astype(jnp.float32)
    sgn_ref[...] = jnp.dot(sgnv, e_ref[...], preferred_element_type=jnp.float32)


def _decode(x):
    return pl.pallas_call(
        _decode_body,
        out_shape=(
            jax.ShapeDtypeStruct((_B, _NCODES), jnp.int32),
            jax.ShapeDtypeStruct((_B, 128), jnp.float32),
        ),
    )(x, _bit_weights(), _sign_expand())


def _perm_table():
    tab = np.zeros((8, 16), np.int32)
    for q in range(4):
        ws = [q * 16 + ll for ll in range(16)]
        tab[q] = [_jmap_word(W) for W in ws]
        tab[4 + q] = [W % 4 for W in ws]
    return jnp.asarray(tab)


def _sc_gather_body(
    codes_hbm, dataw_hbm, perm_hbm, out_hbm, idx_v, perm_v, rows_v, asm_v, sem
):
    cid = lax.axis_index("c")
    sid = lax.axis_index("s")
    wid = sid * 2 + cid
    base = wid * _SPW
    pltpu.sync_copy(codes_hbm.at[pl.ds(base, _SPW)], idx_v)
    pltpu.sync_copy(perm_hbm, perm_v)

    def sample_body(i, carry):
        cp = pltpu.make_async_copy(dataw_hbm.at[idx_v.at[i]], rows_v, sem)
        cp.start()
        cp.wait()

        def t_body(tt, c2):
            t4 = tt * 32
            for k in range(8):
                for q in range(4):
                    vals = plsc.load_gather(
                        rows_v, [perm_v[q, :], perm_v[4 + q, :] + (t4 + 4 * k)]
                    )
                    asm_v[tt * 8 + k, pl.ds(q * 32, 32)] = plsc.bitcast(
                        vals, jnp.uint16
                    )
            return c2

        lax.fori_loop(0, 12, t_body, 0, unroll=False)
        pltpu.sync_copy(asm_v, out_hbm.at[base + i])
        return carry

    lax.fori_loop(0, _SPW, sample_body, 0, unroll=False)


@functools.cache
def _sc_gather():
    return pl.kernel(
        _sc_gather_body,
        out_type=jax.ShapeDtypeStruct((_B, 96, 128), jnp.uint16),
        mesh=plsc.VectorSubcoreMesh(
            core_axis_name="c", subcore_axis_name="s", num_cores=2, num_subcores=16
        ),
        scratch_types=[
            pltpu.VMEM((_SPW, _NCODES), jnp.int32),
            pltpu.VMEM((8, 16), jnp.int32),
            pltpu.VMEM((_NCODES, _ROWW), jnp.int32),
            pltpu.VMEM((96, 128), jnp.uint16),
            pltpu.SemaphoreType.DMA,
        ],
        compiler_params=pltpu.CompilerParams(
            needs_layout_passes=False, use_tc_tiling_on_sc=False
        ),
    )


def _finish_body(g_ref, s_ref, o_ref):
    g = g_ref[...]
    s = s_ref[...][:, None, :]
    core = 0.5 + s * (g - 0.5)
    bs = g.shape[0]
    half = jnp.full((bs, 1, 19, 128), 0.5, jnp.float32)
    o_ref[:, :, 0:19, :] = jnp.broadcast_to(half, (bs, 2, 19, 128))
    o_ref[:, 0, 19:67, :] = core[:, 0:48]
    o_ref[:, 1, 19:67, :] = core[:, 48:96]
    o_ref[:, :, 67:126, :] = jnp.full((bs, 2, 59, 128), 0.5, jnp.float32)


def _finish(gathf, sgn):
    bs = 8
    return pl.pallas_call(
        _finish_body,
        grid=(_B // bs,),
        in_specs=[
            pl.BlockSpec((bs, 96, 128), lambda i: (i, 0, 0)),
            pl.BlockSpec((bs, 128), lambda i: (i, 0)),
        ],
        out_specs=pl.BlockSpec((bs, 2, 126, 128), lambda i: (i, 0, 0, 0)),
        out_shape=jax.ShapeDtypeStruct((_B, 2, 126, 128), jnp.float32),
    )(gathf, sgn)


def kernel(x, data):
    codes, sgn = _decode(x)
    dataw = lax.bitcast_convert_type(
        data.reshape(_LTAB, _ROWW, 2), jnp.int32
    )
    gathw = _sc_gather()(codes, dataw, _perm_table())
    gathf = lax.bitcast_convert_type(gathw, jnp.float16)
    return _finish(gathf.astype(jnp.float32), sgn)


# speedup vs baseline: 3.0205x; 1.0095x over previous
import functools

import jax
import jax.numpy as jnp
import numpy as np
from jax import lax
from jax.experimental import pallas as pl
from jax.experimental.pallas import tpu as pltpu
from jax.experimental.pallas import tpu_sc as plsc

_B = 1024
_NBITS = 22
_NCODES = 23
_LTAB = 131072
_ROWW = 384
_NWORK = 32
_SPW = _B // _NWORK


def _jmap_word(wi):
    g, w = wi // 4, wi % 4
    return g if (g < 7 and w < 2) else g + 7


def _bit_weights():
    wmat = np.zeros((512, _NCODES), np.float32)
    for j in range(_NCODES):
        for i in range(_NBITS):
            wmat[6 + _NBITS * j + i, j] = float(1 << i)
    return jnp.asarray(wmat)


def _sign_expand():
    emat = np.zeros((_NCODES, 128), np.float32)
    for k in range(128):
        emat[_jmap_word(k // 2), k] = 1.0
    return jnp.asarray(emat)


def _decode_body(x_ref, w_ref, e_ref, codes_ref, sgn_ref):
    xf = x_ref[...].astype(jnp.float32)
    vals = jnp.dot(xf, w_ref[...], preferred_element_type=jnp.float32)
    codes_ref[...] = vals.astype(jnp.int32) & (_LTAB - 1)
    sgnv = jnp.where(vals > jnp.float32(_LTAB), -1.0, 1.0).astype(jnp.float32)
    sgn_ref[...] = jnp.dot(sgnv, e_ref[...], preferred_element_type=jnp.float32)


def _decode(x):
    return pl.pallas_call(
        _decode_body,
        out_shape=(
            jax.ShapeDtypeStruct((_B, _NCODES), jnp.int32),
            jax.ShapeDtypeStruct((_B, 128), jnp.float32),
        ),
    )(x, _bit_weights(), _sign_expand())


def _perm_table():
    tab = np.zeros((8, 16), np.int32)
    for q in range(4):
        ws = [q * 16 + ll for ll in range(16)]
        tab[q] = [_jmap_word(W) for W in ws]
        tab[4 + q] = [W % 4 for W in ws]
    return jnp.asarray(tab)


def _sc_gather_body(
    codes_hbm, dataw_hbm, perm_hbm, out_hbm, idx_v, perm_v, rows_v, asm_v, sem
):
    cid = lax.axis_index("c")
    sid = lax.axis_index("s")
    wid = sid * 2 + cid
    base = wid * _SPW
    pltpu.sync_copy(codes_hbm.at[pl.ds(base, _SPW)], idx_v)
    pltpu.sync_copy(perm_hbm, perm_v)

    def sample_body(i, carry):
        cp = pltpu.make_async_copy(dataw_hbm.at[idx_v.at[i]], rows_v, sem)
        cp.start()
        cp.wait()

        def t_body(tt, c2):
            t4 = tt * 32
            for k in range(8):
                for q in range(4):
                    vals = plsc.load_gather(
                        rows_v, [perm_v[q, :], perm_v[4 + q, :] + (t4 + 4 * k)]
                    )
                    asm_v[tt * 8 + k, pl.ds(q * 32, 32)] = plsc.bitcast(
                        vals, jnp.uint16
                    )
            return c2

        lax.fori_loop(0, 12, t_body, 0, unroll=False)
        pltpu.sync_copy(asm_v, out_hbm.at[base + i])
        return carry

    lax.fori_loop(0, _SPW, sample_body, 0, unroll=False)


@functools.cache
def _sc_gather():
    return pl.kernel(
        _sc_gather_body,
        out_type=jax.ShapeDtypeStruct((_B, 96, 128), jnp.uint16),
        mesh=plsc.VectorSubcoreMesh(
            core_axis_name="c", subcore_axis_name="s", num_cores=2, num_subcores=16
        ),
        scratch_types=[
            pltpu.VMEM((_SPW, _NCODES), jnp.int32),
            pltpu.VMEM((8, 16), jnp.int32),
            pltpu.VMEM((_NCODES, _ROWW), jnp.int32),
            pltpu.VMEM((96, 128), jnp.uint16),
            pltpu.SemaphoreType.DMA,
        ],
        compiler_params=pltpu.CompilerParams(
            needs_layout_passes=False, use_tc_tiling_on_sc=False
        ),
    )


def _finish_body(g_ref, s_ref, o_ref):
    gu = g_ref[...].astype(jnp.int32)
    sgnbit = (gu >> 15) << 31
    e = (gu >> 10) & 31
    m = gu & 1023
    normal = lax.bitcast_convert_type(
        sgnbit | ((e + 112) << 23) | (m << 13), jnp.float32
    )
    sub = m.astype(jnp.float32) * jnp.float32(2.0**-24)
    g = jnp.where(e == 0, jnp.where(sgnbit != 0, -sub, sub), normal)
    s = s_ref[...][:, None, :]
    core = 0.5 + s * (g - 0.5)
    bs = g.shape[0]
    half = jnp.full((bs, 1, 19, 128), 0.5, jnp.float32)
    o_ref[:, :, 0:19, :] = jnp.broadcast_to(half, (bs, 2, 19, 128))
    o_ref[:, 0, 19:67, :] = core[:, 0:48]
    o_ref[:, 1, 19:67, :] = core[:, 48:96]
    o_ref[:, :, 67:126, :] = jnp.full((bs, 2, 59, 128), 0.5, jnp.float32)


def _finish(gathf, sgn):
    bs = 8
    return pl.pallas_call(
        _finish_body,
        grid=(_B // bs,),
        in_specs=[
            pl.BlockSpec((bs, 96, 128), lambda i: (i, 0, 0)),
            pl.BlockSpec((bs, 128), lambda i: (i, 0)),
        ],
        out_specs=pl.BlockSpec((bs, 2, 126, 128), lambda i: (i, 0, 0, 0)),
        out_shape=jax.ShapeDtypeStruct((_B, 2, 126, 128), jnp.float32),
    )(gathf, sgn)


def kernel(x, data):
    codes, sgn = _decode(x)
    dataw = lax.bitcast_convert_type(
        data.reshape(_LTAB, _ROWW, 2), jnp.int32
    )
    gathw = _sc_gather()(codes, dataw, _perm_table())
    return _finish(gathw, sgn)
